# Initial kernel scaffold; baseline (speedup 1.0000x reference)
import jax
import jax.numpy as jnp
from jax.experimental import pallas as pl


def kernel(x, edge_index, edge_weight, W, b):
    raise NotImplementedError("write your pallas kernel here")


# speedup vs baseline: 21.9414x; 21.9414x over previous
import functools

import jax
import jax.numpy as jnp
from jax import lax
from jax.experimental import pallas as pl
from jax.experimental.pallas import tpu as pltpu
from jax.experimental.pallas import tpu_sc as plsc

N = 10000
E = 320000
D = 128

NC = 2
NS = 16
NW = NC * NS
L = 16

NPAD = 10240
RPT = NPAD // NS
EPW = E // NW
BLK = 80
NBLK = EPW // BLK

_F32 = jnp.float32
_I32 = jnp.int32


def _mesh():
    return plsc.VectorSubcoreMesh(core_axis_name="c", subcore_axis_name="s")


def _deg_body(dst_hbm, ew_hbm, out_hbm, deg_sh, dbuf2, ebuf, zbuf):
    c = lax.axis_index("c")
    s = lax.axis_index("s")
    w = c * NS + s

    for i in range(RPT // L):
        zbuf[pl.ds(i * L, L)] = jnp.zeros((L,), _F32)
    pltpu.sync_copy(zbuf, deg_sh.at[pl.ds(s * RPT, RPT)])

    pltpu.sync_copy(dst_hbm.at[w], dbuf2)
    pltpu.sync_copy(ew_hbm.at[w], ebuf)
    plsc.subcore_barrier()

    def blk(j, carry):
        pltpu.sync_copy(ebuf.at[pl.ds(j * BLK, BLK)],
                        deg_sh.at[dbuf2.at[j]], add=True)
        return carry

    lax.fori_loop(0, NBLK, blk, 0)
    plsc.subcore_barrier()
    pltpu.sync_copy(deg_sh.at[pl.ds(s * RPT, RPT)],
                    out_hbm.at[c, pl.ds(s * RPT, RPT)])


def _deg_call(dst3, ew2):
    return pl.kernel(
        _deg_body,
        out_type=jax.ShapeDtypeStruct((NC, NPAD), _F32),
        mesh=_mesh(),
        scratch_types=[
            pltpu.VMEM_SHARED((NPAD,), _F32),
            pltpu.VMEM((NBLK, BLK), _I32),
            pltpu.VMEM((EPW,), _F32),
            pltpu.VMEM((RPT,), _F32),
        ],
    )(dst3, ew2)


def _mm_body(x_ref, w_ref, d0_ref, d1_ref, hp_ref, dinv_ref):
    deg = 1.0 + d0_ref[...] + d1_ref[...]
    dinv = lax.rsqrt(deg)
    h = jnp.dot(x_ref[...], w_ref[...],
                preferred_element_type=_F32,
                precision=lax.Precision.HIGHEST)
    hp_ref[...] = h * dinv
    dinv_ref[...] = dinv


_MM_RB = 1024


def _mm_call(x_pad, W, d0, d1):
    return pl.pallas_call(
        _mm_body,
        grid=(NPAD // _MM_RB,),
        in_specs=[
            pl.BlockSpec((_MM_RB, D), lambda i: (i, 0)),
            pl.BlockSpec((D, D), lambda i: (0, 0)),
            pl.BlockSpec((_MM_RB, 1), lambda i: (i, 0)),
            pl.BlockSpec((_MM_RB, 1), lambda i: (i, 0)),
        ],
        out_specs=[
            pl.BlockSpec((_MM_RB, D), lambda i: (i, 0)),
            pl.BlockSpec((_MM_RB, 1), lambda i: (i, 0)),
        ],
        out_shape=[
            jax.ShapeDtypeStruct((NPAD, D), _F32),
            jax.ShapeDtypeStruct((NPAD, 1), _F32),
        ],
    )(x_pad, W, d0, d1)


def _msg_body(hp_hbm, src_hbm, dst_hbm, ew_hbm, out_hbm,
              acc_sh, sbuf, dbuf2, ebuf, rows, zrow, gsem):
    c = lax.axis_index("c")
    s = lax.axis_index("s")
    w = c * NS + s

    for i in range(L):
        for k in range(D // L):
            zrow[i, pl.ds(k * L, L)] = jnp.zeros((L,), _F32)

    def zblk(j, carry):
        pltpu.sync_copy(zrow, acc_sh.at[pl.ds(s * RPT + j * L, L), :])
        return carry

    lax.fori_loop(0, RPT // L, zblk, 0)

    pltpu.sync_copy(src_hbm.at[w], sbuf)
    pltpu.sync_copy(dst_hbm.at[w], dbuf2)
    pltpu.sync_copy(ew_hbm.at[w], ebuf)
    plsc.subcore_barrier()

    def blk(j, carry):
        jbase = j * BLK
        pltpu.async_copy(hp_hbm.at[sbuf.at[pl.ds(jbase, BLK)]],
                         rows, gsem).wait()
        for g in range(BLK // L):
            ew16 = ebuf[pl.ds(jbase + g * L, L)]
            for i in range(L):
                e = g * L + i
                sv = lax.gather(
                    ew16, jnp.full((L, 1), i, dtype=_I32),
                    lax.GatherDimensionNumbers(
                        offset_dims=(), collapsed_slice_dims=(0,),
                        start_index_map=(0,)),
                    slice_sizes=(1,),
                    mode=lax.GatherScatterMode.PROMISE_IN_BOUNDS)
                for k in range(D // L):
                    rows[e, pl.ds(k * L, L)] = rows[e, pl.ds(k * L, L)] * sv
        pltpu.sync_copy(rows, acc_sh.at[dbuf2.at[j]], add=True)
        return carry

    lax.fori_loop(0, NBLK, blk, 0)
    plsc.subcore_barrier()
    pltpu.sync_copy(acc_sh.at[pl.ds(s * RPT, RPT), :],
                    out_hbm.at[c, pl.ds(s * RPT, RPT), :])


def _msg_call(hp, src3, dst3, ew2):
    return pl.kernel(
        _msg_body,
        out_type=jax.ShapeDtypeStruct((NC, NPAD, D), _F32),
        mesh=_mesh(),
        scratch_types=[
            pltpu.VMEM_SHARED((NPAD, D), _F32),
            pltpu.VMEM((EPW,), _I32),
            pltpu.VMEM((NBLK, BLK), _I32),
            pltpu.VMEM((EPW,), _F32),
            pltpu.VMEM((BLK, D), _F32),
            pltpu.VMEM((L, D), _F32),
            pltpu.SemaphoreType.DMA,
        ],
    )(hp, src3, dst3, ew2)


def _out_body(a0_ref, a1_ref, hp_ref, dinv_ref, b_ref, o_ref):
    o_ref[...] = ((a0_ref[...] + a1_ref[...] + hp_ref[...])
                  * dinv_ref[...] + b_ref[...])


def _out_call(a0, a1, hp, dinv, b2):
    return pl.pallas_call(
        _out_body,
        grid=(NPAD // _MM_RB,),
        in_specs=[
            pl.BlockSpec((_MM_RB, D), lambda i: (i, 0)),
            pl.BlockSpec((_MM_RB, D), lambda i: (i, 0)),
            pl.BlockSpec((_MM_RB, D), lambda i: (i, 0)),
            pl.BlockSpec((_MM_RB, 1), lambda i: (i, 0)),
            pl.BlockSpec((1, D), lambda i: (0, 0)),
        ],
        out_specs=pl.BlockSpec((_MM_RB, D), lambda i: (i, 0)),
        out_shape=jax.ShapeDtypeStruct((NPAD, D), _F32),
    )(a0, a1, hp, dinv, b2)


@jax.jit
def kernel(x, edge_index, edge_weight, W, b):
    src = edge_index[0]
    dst = edge_index[1]
    x_pad = jnp.zeros((NPAD, D), _F32).at[:N].set(x.astype(_F32))

    src3 = src.reshape(NW, EPW)
    dst3 = dst.reshape(NW, NBLK, BLK)
    ew2 = edge_weight.reshape(NW, EPW)

    deg2 = _deg_call(dst3, ew2)
    d0 = deg2[0].reshape(NPAD, 1)
    d1 = deg2[1].reshape(NPAD, 1)

    hp, dinv = _mm_call(x_pad, W, d0, d1)
    acc2 = _msg_call(hp, src3, dst3, ew2)
    out = _out_call(acc2[0], acc2[1], hp, dinv, b.reshape(1, D))
    return out[:N]


# speedup vs baseline: 25.5759x; 1.1656x over previous
import functools

import jax
import jax.numpy as jnp
from jax import lax
from jax.experimental import pallas as pl
from jax.experimental.pallas import tpu as pltpu
from jax.experimental.pallas import tpu_sc as plsc

N = 10000
E = 320000
D = 128

NC = 2
NS = 16
NW = NC * NS
L = 16

NPAD = 10240
RPT = NPAD // NS
EPW = E // NW
BLK = 80
NBLK = EPW // BLK
BLK3 = 64
NBLK2 = 160
EPW2 = NBLK2 * BLK3

_F32 = jnp.float32
_I32 = jnp.int32


def _mesh():
    return plsc.VectorSubcoreMesh(core_axis_name="c", subcore_axis_name="s")


def _deg_body(dst_hbm, ew_hbm, out_hbm, deg_sh, dbuf2, ebuf, zbuf):
    c = lax.axis_index("c")
    s = lax.axis_index("s")
    w = c * NS + s

    for i in range(RPT // L):
        zbuf[pl.ds(i * L, L)] = jnp.zeros((L,), _F32)
    pltpu.sync_copy(zbuf, deg_sh.at[pl.ds(s * RPT, RPT)])

    pltpu.sync_copy(dst_hbm.at[w], dbuf2)
    pltpu.sync_copy(ew_hbm.at[w], ebuf)
    plsc.subcore_barrier()

    def blk(j, carry):
        pltpu.sync_copy(ebuf.at[pl.ds(j * BLK, BLK)],
                        deg_sh.at[dbuf2.at[j]], add=True)
        return carry

    lax.fori_loop(0, NBLK, blk, 0)
    plsc.subcore_barrier()
    pltpu.sync_copy(deg_sh.at[pl.ds(s * RPT, RPT)],
                    out_hbm.at[c, pl.ds(s * RPT, RPT)])


def _deg_call(dst3, ew2):
    return pl.kernel(
        _deg_body,
        out_type=jax.ShapeDtypeStruct((NC, NPAD), _F32),
        mesh=_mesh(),
        scratch_types=[
            pltpu.VMEM_SHARED((NPAD,), _F32),
            pltpu.VMEM((NBLK, BLK), _I32),
            pltpu.VMEM((EPW,), _F32),
            pltpu.VMEM((RPT,), _F32),
        ],
    )(dst3, ew2)


def _mm_body(x_ref, w_ref, d0_ref, d1_ref, hp_ref, dinv_ref):
    deg = 1.0 + d0_ref[...] + d1_ref[...]
    dinv = lax.rsqrt(deg)
    h = jnp.dot(x_ref[...], w_ref[...],
                preferred_element_type=_F32,
                precision=lax.Precision.HIGHEST)
    hp_ref[...] = h * dinv
    dinv_ref[...] = dinv


_MM_RB = 1024


def _mm_call(x_pad, W, d0, d1):
    return pl.pallas_call(
        _mm_body,
        grid=(NPAD // _MM_RB,),
        in_specs=[
            pl.BlockSpec((_MM_RB, D), lambda i: (i, 0)),
            pl.BlockSpec((D, D), lambda i: (0, 0)),
            pl.BlockSpec((_MM_RB, 1), lambda i: (i, 0)),
            pl.BlockSpec((_MM_RB, 1), lambda i: (i, 0)),
        ],
        out_specs=[
            pl.BlockSpec((_MM_RB, D), lambda i: (i, 0)),
            pl.BlockSpec((_MM_RB, 1), lambda i: (i, 0)),
        ],
        out_shape=[
            jax.ShapeDtypeStruct((NPAD, D), _F32),
            jax.ShapeDtypeStruct((NPAD, 1), _F32),
        ],
    )(x_pad, W, d0, d1)


NSLOT = 4
RING = 2


def _msg_body(pk_hbm, ew_hbm, hp_hbm, out_hbm,
              acc_sh, r0, r1, ebuf, pb0, pb1, pb2, pb3,
              g0, g1, s0, s1, p0, p1, p2, p3):
    c = lax.axis_index("c")
    s = lax.axis_index("s")
    w = c * NS + s
    rows = (r0, r1)
    gsem = (g0, g1)
    ssem = (s0, s1)
    pbuf = (pb0, pb1, pb2, pb3)
    psem = (p0, p1, p2, p3)

    for i in range(L):
        for k in range(D // L):
            r0[i, pl.ds(k * L, L)] = jnp.zeros((L,), _F32)

    def zblk(j, carry):
        pltpu.sync_copy(r0.at[pl.ds(0, L), :],
                        acc_sh.at[pl.ds(s * RPT + j * L, L), :])
        return carry

    lax.fori_loop(0, RPT // L, zblk, 0)

    pltpu.sync_copy(ew_hbm.at[w], ebuf)
    plsc.subcore_barrier()

    def _load(j, slot):
        jj = jnp.where(j < NBLK2, j, 0)
        pltpu.async_copy(pk_hbm.at[w, jj], pbuf[slot], psem[slot])

    def _load_wait(j, slot):
        pltpu.make_async_copy(pk_hbm.at[w, 0], pbuf[slot],
                              psem[slot]).wait()

    def _gather(slot, b):
        pltpu.async_copy(hp_hbm.at[pbuf[slot].at[0]], rows[b], gsem[b])

    def _gather_wait(slot, b):
        pltpu.make_async_copy(hp_hbm.at[pbuf[slot].at[0]], rows[b],
                              gsem[b]).wait()

    def _scatter(slot, b):
        pltpu.async_copy(rows[b], acc_sh.at[pbuf[slot].at[1]],
                         ssem[b], add=True)

    def _scatter_wait(slot, b):
        pltpu.make_async_copy(rows[b], acc_sh.at[pbuf[slot].at[1]],
                              ssem[b]).wait()

    def _scale(slot, b, j):
        rb = rows[b]
        for g in range(BLK3 // L):
            ew16 = ebuf[pl.ds(j * BLK3 + g * L, L)]
            for i in range(L):
                e = g * L + i
                sv = lax.gather(
                    ew16, jnp.full((L, 1), i, dtype=_I32),
                    lax.GatherDimensionNumbers(
                        offset_dims=(), collapsed_slice_dims=(0,),
                        start_index_map=(0,)),
                    slice_sizes=(1,),
                    mode=lax.GatherScatterMode.PROMISE_IN_BOUNDS)
                for k in range(D // L):
                    rb[e, pl.ds(k * L, L)] = rb[e, pl.ds(k * L, L)] * sv

    _load(0, 0)
    _load(1, 1)
    _load_wait(0, 0)
    _gather(0, 0)

    def outer(gi, carry):
        for b4 in range(NSLOT):
            j = gi * NSLOT + b4
            b = b4 % RING
            bn = (b + 1) % RING
            sl = b4
            sl1 = (b4 + 1) % NSLOT
            sl2 = (b4 + 2) % NSLOT
            sl3 = (b4 + 3) % NSLOT
            @pl.when(j > 0)
            def _():
                _scatter_wait(sl3, bn)
            _load(j + 2, sl2)
            _load_wait(j + 1, sl1)
            _gather(sl1, bn)
            _gather_wait(sl, b)
            _scale(sl, b, j)
            _scatter(sl, b)
        return carry

    lax.fori_loop(0, NBLK2 // NSLOT, outer, 0)

    _load_wait(0, 1)
    _gather_wait(0, 0)
    _scatter_wait(3, 1)

    plsc.subcore_barrier()
    pltpu.sync_copy(acc_sh.at[pl.ds(s * RPT, RPT), :],
                    out_hbm.at[c, pl.ds(s * RPT, RPT), :])


def _msg_call(hp, pk4, ewp):
    return pl.kernel(
        _msg_body,
        out_type=jax.ShapeDtypeStruct((NC, NPAD, D), _F32),
        mesh=_mesh(),
        scratch_types=[
            pltpu.VMEM_SHARED((NPAD, D), _F32),
            pltpu.VMEM((BLK3, D), _F32),
            pltpu.VMEM((BLK3, D), _F32),
            pltpu.VMEM((EPW2,), _F32),
            pltpu.VMEM((2, BLK3), _I32),
            pltpu.VMEM((2, BLK3), _I32),
            pltpu.VMEM((2, BLK3), _I32),
            pltpu.VMEM((2, BLK3), _I32),
            pltpu.SemaphoreType.DMA,
            pltpu.SemaphoreType.DMA,
            pltpu.SemaphoreType.DMA,
            pltpu.SemaphoreType.DMA,
            pltpu.SemaphoreType.DMA,
            pltpu.SemaphoreType.DMA,
            pltpu.SemaphoreType.DMA,
            pltpu.SemaphoreType.DMA,
        ],
    )(pk4, ewp, hp)


def _out_body(a0_ref, a1_ref, hp_ref, dinv_ref, b_ref, o_ref):
    o_ref[...] = ((a0_ref[...] + a1_ref[...] + hp_ref[...])
                  * dinv_ref[...] + b_ref[...])


def _out_call(a0, a1, hp, dinv, b2):
    return pl.pallas_call(
        _out_body,
        grid=(NPAD // _MM_RB,),
        in_specs=[
            pl.BlockSpec((_MM_RB, D), lambda i: (i, 0)),
            pl.BlockSpec((_MM_RB, D), lambda i: (i, 0)),
            pl.BlockSpec((_MM_RB, D), lambda i: (i, 0)),
            pl.BlockSpec((_MM_RB, 1), lambda i: (i, 0)),
            pl.BlockSpec((1, D), lambda i: (0, 0)),
        ],
        out_specs=pl.BlockSpec((_MM_RB, D), lambda i: (i, 0)),
        out_shape=jax.ShapeDtypeStruct((NPAD, D), _F32),
    )(a0, a1, hp, dinv, b2)


@jax.jit
def kernel(x, edge_index, edge_weight, W, b):
    src = edge_index[0]
    dst = edge_index[1]
    x_pad = jnp.zeros((NPAD, D), _F32).at[:N].set(x.astype(_F32))

    dst3 = dst.reshape(NW, NBLK, BLK)
    ew2 = edge_weight.reshape(NW, EPW)

    npad_e = EPW2 - EPW
    pad_idx = (jnp.arange(npad_e, dtype=_I32) % N)
    srcp = jnp.concatenate(
        [src.reshape(NW, EPW),
         jnp.broadcast_to(pad_idx, (NW, npad_e))], axis=1)
    dstp = jnp.concatenate(
        [dst.reshape(NW, EPW),
         jnp.broadcast_to(pad_idx, (NW, npad_e))], axis=1)
    ewp = jnp.concatenate(
        [ew2, jnp.zeros((NW, npad_e), _F32)], axis=1)
    pk4 = jnp.stack(
        [srcp.reshape(NW, NBLK2, BLK3),
         dstp.reshape(NW, NBLK2, BLK3)],
        axis=2)

    deg2 = _deg_call(dst3, ew2)
    d0 = deg2[0].reshape(NPAD, 1)
    d1 = deg2[1].reshape(NPAD, 1)

    hp, dinv = _mm_call(x_pad, W, d0, d1)
    acc2 = _msg_call(hp, pk4, ewp)
    out = _out_call(acc2[0], acc2[1], hp, dinv, b.reshape(1, D))
    return out[:N]


# speedup vs baseline: 26.0803x; 1.0197x over previous
import functools

import jax
import jax.numpy as jnp
from jax import lax
from jax.experimental import pallas as pl
from jax.experimental.pallas import tpu as pltpu
from jax.experimental.pallas import tpu_sc as plsc

N = 10000
E = 320000
D = 128

NC = 2
NS = 16
NW = NC * NS
L = 16

NPAD = 10240
RPT = NPAD // NS
EPW = E // NW
BLK = 80
NBLK = EPW // BLK
BLK3 = 64
NBLK2 = 162
EPW2 = NBLK2 * BLK3

_F32 = jnp.float32
_I32 = jnp.int32


def _mesh():
    return plsc.VectorSubcoreMesh(core_axis_name="c", subcore_axis_name="s")


def _deg_body(dst_hbm, ew_hbm, out_hbm, deg_sh, dbuf2, ebuf, zbuf):
    c = lax.axis_index("c")
    s = lax.axis_index("s")
    w = c * NS + s

    for i in range(RPT // L):
        zbuf[pl.ds(i * L, L)] = jnp.zeros((L,), _F32)
    pltpu.sync_copy(zbuf, deg_sh.at[pl.ds(s * RPT, RPT)])

    pltpu.sync_copy(dst_hbm.at[w], dbuf2)
    pltpu.sync_copy(ew_hbm.at[w], ebuf)
    plsc.subcore_barrier()

    def blk(j, carry):
        pltpu.sync_copy(ebuf.at[pl.ds(j * BLK, BLK)],
                        deg_sh.at[dbuf2.at[j]], add=True)
        return carry

    lax.fori_loop(0, NBLK, blk, 0)
    plsc.subcore_barrier()
    pltpu.sync_copy(deg_sh.at[pl.ds(s * RPT, RPT)],
                    out_hbm.at[c, pl.ds(s * RPT, RPT)])


def _deg_call(dst3, ew2):
    return pl.kernel(
        _deg_body,
        out_type=jax.ShapeDtypeStruct((NC, NPAD), _F32),
        mesh=_mesh(),
        scratch_types=[
            pltpu.VMEM_SHARED((NPAD,), _F32),
            pltpu.VMEM((NBLK, BLK), _I32),
            pltpu.VMEM((EPW,), _F32),
            pltpu.VMEM((RPT,), _F32),
        ],
    )(dst3, ew2)


def _mm_body(x_ref, w_ref, d0_ref, d1_ref, hp_ref, dinv_ref):
    deg = 1.0 + d0_ref[...] + d1_ref[...]
    dinv = lax.rsqrt(deg)
    h = jnp.dot(x_ref[...], w_ref[...],
                preferred_element_type=_F32,
                precision=lax.Precision.HIGHEST)
    hp_ref[...] = h * dinv
    dinv_ref[...] = dinv


_MM_RB = 1024


def _mm_call(x_pad, W, d0, d1):
    return pl.pallas_call(
        _mm_body,
        grid=(NPAD // _MM_RB,),
        in_specs=[
            pl.BlockSpec((_MM_RB, D), lambda i: (i, 0)),
            pl.BlockSpec((D, D), lambda i: (0, 0)),
            pl.BlockSpec((_MM_RB, 1), lambda i: (i, 0)),
            pl.BlockSpec((_MM_RB, 1), lambda i: (i, 0)),
        ],
        out_specs=[
            pl.BlockSpec((_MM_RB, D), lambda i: (i, 0)),
            pl.BlockSpec((_MM_RB, 1), lambda i: (i, 0)),
        ],
        out_shape=[
            jax.ShapeDtypeStruct((NPAD, D), _F32),
            jax.ShapeDtypeStruct((NPAD, 1), _F32),
        ],
    )(x_pad, W, d0, d1)


NSLOT = 6
RING = 3


def _msg_body(pk_hbm, ew_hbm, hp_hbm, out_hbm,
              acc_sh, r0, r1, r2, ebuf, pb0, pb1, pb2, pb3, pb4, pb5,
              g0, g1, g2, sc0, sc1, sc2, p0, p1, p2, p3, p4, p5):
    c = lax.axis_index("c")
    s = lax.axis_index("s")
    w = c * NS + s
    rows = (r0, r1, r2)
    gsem = (g0, g1, g2)
    ssem = (sc0, sc1, sc2)
    pbuf = (pb0, pb1, pb2, pb3, pb4, pb5)
    psem = (p0, p1, p2, p3, p4, p5)

    for i in range(L):
        for k in range(D // L):
            r0[i, pl.ds(k * L, L)] = jnp.zeros((L,), _F32)

    def zblk(j, carry):
        pltpu.sync_copy(r0.at[pl.ds(0, L), :],
                        acc_sh.at[pl.ds(s * RPT + j * L, L), :])
        return carry

    lax.fori_loop(0, RPT // L, zblk, 0)

    pltpu.sync_copy(ew_hbm.at[w], ebuf)
    plsc.subcore_barrier()

    def _load(j, slot):
        jj = jnp.where(j < NBLK2, j, 0)
        pltpu.async_copy(pk_hbm.at[w, jj], pbuf[slot], psem[slot])

    def _load_wait(j, slot):
        pltpu.make_async_copy(pk_hbm.at[w, 0], pbuf[slot],
                              psem[slot]).wait()

    def _gather(slot, b):
        pltpu.async_copy(hp_hbm.at[pbuf[slot].at[0]], rows[b], gsem[b])

    def _gather_wait(slot, b):
        pltpu.make_async_copy(hp_hbm.at[pbuf[slot].at[0]], rows[b],
                              gsem[b]).wait()

    def _scatter(slot, b):
        pltpu.async_copy(rows[b], acc_sh.at[pbuf[slot].at[1]],
                         ssem[b], add=True)

    def _scatter_wait(slot, b):
        pltpu.make_async_copy(rows[b], acc_sh.at[pbuf[slot].at[1]],
                              ssem[b]).wait()

    def _scale(slot, b, j):
        rb = rows[b]
        for g in range(BLK3 // L):
            ew16 = ebuf[pl.ds(j * BLK3 + g * L, L)]
            for i in range(L):
                e = g * L + i
                sv = lax.gather(
                    ew16, jnp.full((L, 1), i, dtype=_I32),
                    lax.GatherDimensionNumbers(
                        offset_dims=(), collapsed_slice_dims=(0,),
                        start_index_map=(0,)),
                    slice_sizes=(1,),
                    mode=lax.GatherScatterMode.PROMISE_IN_BOUNDS)
                for k in range(D // L):
                    rb[e, pl.ds(k * L, L)] = rb[e, pl.ds(k * L, L)] * sv

    _load(0, 0)
    _load(1, 1)
    _load_wait(0, 0)
    _gather(0, 0)

    def outer(gi, carry):
        for u in range(NSLOT):
            j = gi * NSLOT + u
            b = u % RING
            b1 = (u + 1) % RING
            sl = u
            sl1 = (u + 1) % NSLOT
            sl2 = (u + 2) % NSLOT
            slm2 = (u + NSLOT - 2) % NSLOT
            bm2 = (u + RING - 2) % RING
            @pl.when(j > 1)
            def _():
                _scatter_wait(slm2, bm2)
            _load(j + 2, sl2)
            _load_wait(j + 1, sl1)
            _gather(sl1, b1)
            _gather_wait(sl, b)
            _scale(sl, b, j)
            _scatter(sl, b)
        return carry

    lax.fori_loop(0, NBLK2 // NSLOT, outer, 0)

    _load_wait(0, 1)
    _gather_wait(0, 0)
    _scatter_wait((NBLK2 - 2) % NSLOT, (NBLK2 - 2) % RING)
    _scatter_wait((NBLK2 - 1) % NSLOT, (NBLK2 - 1) % RING)

    plsc.subcore_barrier()
    pltpu.sync_copy(acc_sh.at[pl.ds(s * RPT, RPT), :],
                    out_hbm.at[c, pl.ds(s * RPT, RPT), :])


def _msg_call(hp, pk4, ewp):
    return pl.kernel(
        _msg_body,
        out_type=jax.ShapeDtypeStruct((NC, NPAD, D), _F32),
        mesh=_mesh(),
        scratch_types=(
            [pltpu.VMEM_SHARED((NPAD, D), _F32)]
            + [pltpu.VMEM((BLK3, D), _F32)] * RING
            + [pltpu.VMEM((EPW2,), _F32)]
            + [pltpu.VMEM((2, BLK3), _I32)] * NSLOT
            + [pltpu.SemaphoreType.DMA] * (2 * RING + NSLOT)
        ),
    )(pk4, ewp, hp)


def _out_body(a0_ref, a1_ref, hp_ref, dinv_ref, b_ref, o_ref):
    o_ref[...] = ((a0_ref[...] + a1_ref[...] + hp_ref[...])
                  * dinv_ref[...] + b_ref[...])


def _out_call(a0, a1, hp, dinv, b2):
    return pl.pallas_call(
        _out_body,
        grid=(NPAD // _MM_RB,),
        in_specs=[
            pl.BlockSpec((_MM_RB, D), lambda i: (i, 0)),
            pl.BlockSpec((_MM_RB, D), lambda i: (i, 0)),
            pl.BlockSpec((_MM_RB, D), lambda i: (i, 0)),
            pl.BlockSpec((_MM_RB, 1), lambda i: (i, 0)),
            pl.BlockSpec((1, D), lambda i: (0, 0)),
        ],
        out_specs=pl.BlockSpec((_MM_RB, D), lambda i: (i, 0)),
        out_shape=jax.ShapeDtypeStruct((NPAD, D), _F32),
    )(a0, a1, hp, dinv, b2)


@jax.jit
def kernel(x, edge_index, edge_weight, W, b):
    src = edge_index[0]
    dst = edge_index[1]
    x_pad = jnp.zeros((NPAD, D), _F32).at[:N].set(x.astype(_F32))

    dst3 = dst.reshape(NW, NBLK, BLK)
    ew2 = edge_weight.reshape(NW, EPW)

    npad_e = EPW2 - EPW
    pad_idx = (jnp.arange(npad_e, dtype=_I32) % N)
    srcp = jnp.concatenate(
        [src.reshape(NW, EPW),
         jnp.broadcast_to(pad_idx, (NW, npad_e))], axis=1)
    dstp = jnp.concatenate(
        [dst.reshape(NW, EPW),
         jnp.broadcast_to(pad_idx, (NW, npad_e))], axis=1)
    ewp = jnp.concatenate(
        [ew2, jnp.zeros((NW, npad_e), _F32)], axis=1)
    pk4 = jnp.stack(
        [srcp.reshape(NW, NBLK2, BLK3),
         dstp.reshape(NW, NBLK2, BLK3)],
        axis=2)

    deg2 = _deg_call(dst3, ew2)
    d0 = deg2[0].reshape(NPAD, 1)
    d1 = deg2[1].reshape(NPAD, 1)

    hp, dinv = _mm_call(x_pad, W, d0, d1)
    acc2 = _msg_call(hp, pk4, ewp)
    out = _out_call(acc2[0], acc2[1], hp, dinv, b.reshape(1, D))
    return out[:N]


# speedup vs baseline: 26.6398x; 1.0215x over previous
import functools

import jax
import jax.numpy as jnp
from jax import lax
from jax.experimental import pallas as pl
from jax.experimental.pallas import tpu as pltpu
from jax.experimental.pallas import tpu_sc as plsc

N = 10000
E = 320000
D = 128

NC = 2
NS = 16
NW = NC * NS
L = 16

NPAD = 10240
RPT = NPAD // NS
EPW = E // NW
BLK = 80
NBLK = EPW // BLK
BLK3 = 80
NBLK2 = 126
EPW2 = NBLK2 * BLK3

_F32 = jnp.float32
_I32 = jnp.int32


def _mesh():
    return plsc.VectorSubcoreMesh(core_axis_name="c", subcore_axis_name="s")


def _deg_body(dst_hbm, ew_hbm, out_hbm, deg_sh, dbuf2, ebuf, zbuf):
    c = lax.axis_index("c")
    s = lax.axis_index("s")
    w = c * NS + s

    for i in range(RPT // L):
        zbuf[pl.ds(i * L, L)] = jnp.zeros((L,), _F32)
    pltpu.sync_copy(zbuf, deg_sh.at[pl.ds(s * RPT, RPT)])

    pltpu.sync_copy(dst_hbm.at[w], dbuf2)
    pltpu.sync_copy(ew_hbm.at[w], ebuf)
    plsc.subcore_barrier()

    def blk(j, carry):
        pltpu.sync_copy(ebuf.at[pl.ds(j * BLK, BLK)],
                        deg_sh.at[dbuf2.at[j]], add=True)
        return carry

    lax.fori_loop(0, NBLK, blk, 0)
    plsc.subcore_barrier()
    pltpu.sync_copy(deg_sh.at[pl.ds(s * RPT, RPT)],
                    out_hbm.at[c, pl.ds(s * RPT, RPT)])


def _deg_call(dst3, ew2):
    return pl.kernel(
        _deg_body,
        out_type=jax.ShapeDtypeStruct((NC, NPAD), _F32),
        mesh=_mesh(),
        scratch_types=[
            pltpu.VMEM_SHARED((NPAD,), _F32),
            pltpu.VMEM((NBLK, BLK), _I32),
            pltpu.VMEM((EPW,), _F32),
            pltpu.VMEM((RPT,), _F32),
        ],
    )(dst3, ew2)


def _mm_body(x_ref, w_ref, d0_ref, d1_ref, hp_ref, dinv_ref):
    deg = 1.0 + d0_ref[...] + d1_ref[...]
    dinv = lax.rsqrt(deg)
    h = jnp.dot(x_ref[...], w_ref[...],
                preferred_element_type=_F32,
                precision=lax.Precision.HIGHEST)
    hp_ref[...] = h * dinv
    dinv_ref[...] = dinv


_MM_RB = 1024


def _mm_call(x_pad, W, d0, d1):
    return pl.pallas_call(
        _mm_body,
        grid=(NPAD // _MM_RB,),
        in_specs=[
            pl.BlockSpec((_MM_RB, D), lambda i: (i, 0)),
            pl.BlockSpec((D, D), lambda i: (0, 0)),
            pl.BlockSpec((_MM_RB, 1), lambda i: (i, 0)),
            pl.BlockSpec((_MM_RB, 1), lambda i: (i, 0)),
        ],
        out_specs=[
            pl.BlockSpec((_MM_RB, D), lambda i: (i, 0)),
            pl.BlockSpec((_MM_RB, 1), lambda i: (i, 0)),
        ],
        out_shape=[
            jax.ShapeDtypeStruct((NPAD, D), _F32),
            jax.ShapeDtypeStruct((NPAD, 1), _F32),
        ],
    )(x_pad, W, d0, d1)


NSLOT = 6
RING = 3


def _msg_body(pk_hbm, ew_hbm, hp_hbm, out_hbm,
              acc_sh, r0, r1, r2, ebuf,
              pb0, pb1, pb2, pb3, pb4, pb5,
              g0, g1, g2, sc0, sc1, sc2, p0, p1, p2, p3, p4, p5):
    c = lax.axis_index("c")
    s = lax.axis_index("s")
    w = c * NS + s
    rows = (r0, r1, r2)
    gsem = (g0, g1, g2)
    ssem = (sc0, sc1, sc2)
    pbuf = (pb0, pb1, pb2, pb3, pb4, pb5)
    psem = (p0, p1, p2, p3, p4, p5)

    for i in range(L):
        for k in range(D // L):
            r0[i, pl.ds(k * L, L)] = jnp.zeros((L,), _F32)

    def zblk(j, carry):
        pltpu.sync_copy(r0.at[pl.ds(0, L), :],
                        acc_sh.at[pl.ds(s * RPT + j * L, L), :])
        return carry

    lax.fori_loop(0, RPT // L, zblk, 0)

    pltpu.sync_copy(ew_hbm.at[w], ebuf)
    plsc.subcore_barrier()

    def _load(j, slot):
        jj = jnp.where(j < NBLK2, j, 0)
        pltpu.async_copy(pk_hbm.at[w, jj], pbuf[slot], psem[slot])

    def _load_wait(j, slot):
        pltpu.make_async_copy(pk_hbm.at[w, 0], pbuf[slot],
                              psem[slot]).wait()

    def _gather(slot, b):
        pltpu.async_copy(hp_hbm.at[pbuf[slot].at[0]], rows[b], gsem[b])

    def _gather_wait(slot, b):
        pltpu.make_async_copy(hp_hbm.at[pbuf[slot].at[0]], rows[b],
                              gsem[b]).wait()

    def _scatter(slot, b):
        pltpu.async_copy(rows[b], acc_sh.at[pbuf[slot].at[1]],
                         ssem[b], add=True)

    def _scatter_wait(slot, b):
        pltpu.make_async_copy(rows[b], acc_sh.at[pbuf[slot].at[1]],
                              ssem[b]).wait()

    def _scale(slot, b, j):
        rb = rows[b]
        for g in range(BLK3 // L):
            ew16 = ebuf[pl.ds(j * BLK3 + g * L, L)]
            for i in range(L):
                e = g * L + i
                sv = lax.gather(
                    ew16, jnp.full((L, 1), i, dtype=_I32),
                    lax.GatherDimensionNumbers(
                        offset_dims=(), collapsed_slice_dims=(0,),
                        start_index_map=(0,)),
                    slice_sizes=(1,),
                    mode=lax.GatherScatterMode.PROMISE_IN_BOUNDS)
                for k in range(D // L):
                    rb[e, pl.ds(k * L, L)] = rb[e, pl.ds(k * L, L)] * sv

    _load(0, 0)
    _load(1, 1)
    _load_wait(0, 0)
    _gather(0, 0)

    def outer(gi, carry):
        for u in range(NSLOT):
            j = gi * NSLOT + u
            b = u % RING
            b1 = (u + 1) % RING
            sl = u
            sl1 = (u + 1) % NSLOT
            sl2 = (u + 2) % NSLOT
            slm2 = (u + NSLOT - 2) % NSLOT
            bm2 = (u + RING - 2) % RING
            @pl.when(j > 1)
            def _():
                _scatter_wait(slm2, bm2)
            _load(j + 2, sl2)
            _load_wait(j + 1, sl1)
            _gather(sl1, b1)
            _gather_wait(sl, b)
            _scale(sl, b, j)
            _scatter(sl, b)
        return carry

    lax.fori_loop(0, NBLK2 // NSLOT, outer, 0)

    _load_wait(0, 1)
    _gather_wait(0, 0)
    _scatter_wait((NBLK2 - 2) % NSLOT, (NBLK2 - 2) % RING)
    _scatter_wait((NBLK2 - 1) % NSLOT, (NBLK2 - 1) % RING)

    plsc.subcore_barrier()
    pltpu.sync_copy(acc_sh.at[pl.ds(s * RPT, RPT), :],
                    out_hbm.at[c, pl.ds(s * RPT, RPT), :])


def _msg_call(hp, pk4, ewp):
    return pl.kernel(
        _msg_body,
        out_type=jax.ShapeDtypeStruct((NC, NPAD, D), _F32),
        mesh=_mesh(),
        scratch_types=(
            [pltpu.VMEM_SHARED((NPAD, D), _F32)]
            + [pltpu.VMEM((BLK3, D), _F32)] * RING
            + [pltpu.VMEM((EPW2,), _F32)]
            + [pltpu.VMEM((2, BLK3), _I32)] * NSLOT
            + [pltpu.SemaphoreType.DMA] * (2 * RING + NSLOT)
        ),
    )(pk4, ewp, hp)


def _out_body(a0_ref, a1_ref, hp_ref, dinv_ref, b_ref, o_ref):
    o_ref[...] = ((a0_ref[...] + a1_ref[...] + hp_ref[...])
                  * dinv_ref[...] + b_ref[...])


def _out_call(a0, a1, hp, dinv, b2):
    return pl.pallas_call(
        _out_body,
        grid=(NPAD // _MM_RB,),
        in_specs=[
            pl.BlockSpec((_MM_RB, D), lambda i: (i, 0)),
            pl.BlockSpec((_MM_RB, D), lambda i: (i, 0)),
            pl.BlockSpec((_MM_RB, D), lambda i: (i, 0)),
            pl.BlockSpec((_MM_RB, 1), lambda i: (i, 0)),
            pl.BlockSpec((1, D), lambda i: (0, 0)),
        ],
        out_specs=pl.BlockSpec((_MM_RB, D), lambda i: (i, 0)),
        out_shape=jax.ShapeDtypeStruct((NPAD, D), _F32),
    )(a0, a1, hp, dinv, b2)


@jax.jit
def kernel(x, edge_index, edge_weight, W, b):
    src = edge_index[0]
    dst = edge_index[1]
    x_pad = jnp.zeros((NPAD, D), _F32).at[:N].set(x.astype(_F32))

    dst3 = dst.reshape(NW, NBLK, BLK)
    ew2 = edge_weight.reshape(NW, EPW)

    npad_e = EPW2 - EPW
    pad_idx = (jnp.arange(npad_e, dtype=_I32) % N)
    srcp = jnp.concatenate(
        [src.reshape(NW, EPW),
         jnp.broadcast_to(pad_idx, (NW, npad_e))], axis=1)
    dstp = jnp.concatenate(
        [dst.reshape(NW, EPW),
         jnp.broadcast_to(pad_idx, (NW, npad_e))], axis=1)
    ewp = jnp.concatenate(
        [ew2, jnp.zeros((NW, npad_e), _F32)], axis=1)
    pk4 = jnp.stack(
        [srcp.reshape(NW, NBLK2, BLK3),
         dstp.reshape(NW, NBLK2, BLK3)],
        axis=2)

    deg2 = _deg_call(dst3, ew2)
    d0 = deg2[0].reshape(NPAD, 1)
    d1 = deg2[1].reshape(NPAD, 1)

    hp, dinv = _mm_call(x_pad, W, d0, d1)
    acc2 = _msg_call(hp, pk4, ewp)
    out = _out_call(acc2[0], acc2[1], hp, dinv, b.reshape(1, D))
    return out[:N]
